# baseline (speedup 1.0000x reference)
import jax
import jax.numpy as jnp
from jax import lax
from jax.experimental import pallas as pl
from jax.experimental.pallas import tpu as pltpu
from jax.experimental.pallas import tpu_sc as plsc

N_NODES = 10000
N_EDGES = 320000
D = 128

NC = 2
NS = 16
NW = NC * NS
EPW = N_EDGES // NW
CHUNK = 128
HALF = CHUNK // 2
NCHUNK = EPW // CHUNK
TAIL = EPW - NCHUNK * CHUNK
N_PAD = 10240
ROWS_PT = N_PAD // NS
ZROWS = HALF
CNT_PAD = N_PAD
CNT_PT = CNT_PAD // NS


def _shuffled_bf16(a):
  n = a.shape[0]
  shuf = (a.reshape(n, 4, 2, 16).swapaxes(2, 3).reshape(n, D // 2, 2)
          .astype(jnp.bfloat16))
  return jax.lax.bitcast_convert_type(shuf, jnp.int32)


def _make_sc_agg(with_cnt: bool):
  mesh = plsc.VectorSubcoreMesh(
      core_axis_name="c", subcore_axis_name="s", num_cores=NC, num_subcores=NS)
  out_type = [jax.ShapeDtypeStruct((NC, N_PAD, D), jnp.float32)]
  scratch = [
      pltpu.VMEM((NCHUNK * CHUNK,), jnp.int32),
      pltpu.VMEM((3, 2, HALF), jnp.int32),
      pltpu.VMEM((2, CHUNK, D // 2), jnp.int32),
      pltpu.VMEM((2, HALF, D), jnp.float32),
      pltpu.VMEM((TAIL,), jnp.int32),
      pltpu.VMEM((TAIL,), jnp.int32),
      pltpu.VMEM((TAIL, D // 2), jnp.int32),
      pltpu.VMEM((TAIL, D), jnp.float32),
      pltpu.VMEM_SHARED((N_PAD, D), jnp.float32),
      pltpu.SemaphoreType.DMA,
      pltpu.SemaphoreType.DMA,
      pltpu.SemaphoreType.DMA,
      pltpu.SemaphoreType.DMA,
      pltpu.SemaphoreType.DMA,
      pltpu.SemaphoreType.DMA,
      pltpu.SemaphoreType.DMA,
      pltpu.SemaphoreType.DMA,
  ]
  if with_cnt:
    out_type.append(jax.ShapeDtypeStruct((NC * CNT_PAD,), jnp.float32))
    scratch += [
        pltpu.VMEM((HALF,), jnp.float32),
        pltpu.VMEM((TAIL,), jnp.float32),
        pltpu.VMEM((CNT_PT,), jnp.float32),
        pltpu.VMEM_SHARED((CNT_PAD,), jnp.float32),
        pltpu.SemaphoreType.DMA,
        pltpu.SemaphoreType.DMA,
    ]

  def body(xbf_hbm, src_hbm, dst_hbm, sum_hbm, *rest):
    if with_cnt:
      (cnt_hbm, src_all, dstb, rows_bf, fb, srct, dstt, tailbf, rowst, acc_s,
       isem, gsem0, gsem1, fsem0, fsem1, dsem0, dsem1, dsem2,
       ones_v, onest, zcnt_v, cnt_s, csem0, csem1) = rest
    else:
      (src_all, dstb, rows_bf, fb, srct, dstt, tailbf, rowst, acc_s,
       isem, gsem0, gsem1, fsem0, fsem1, dsem0, dsem1, dsem2) = rest
    gsem = (gsem0, gsem1)
    fsem = (fsem0, fsem1)
    dsem = (dsem0, dsem1, dsem2)
    if with_cnt:
      csem = (csem0, csem1)
    zbuf_v = fb.at[0]
    cid = lax.axis_index("c")
    sid = lax.axis_index("s")
    wid = sid * NC + cid
    e0 = wid * EPW

    def pf_dst(i, j):
      for h in (0, 1):
        pltpu.async_copy(dst_hbm.at[pl.ds(e0 + i * CHUNK + h * HALF, HALF)],
                         dstb.at[j, h], dsem[j])

    def wait_dst(i, j):
      for h in (0, 1):
        pltpu.make_async_copy(
            dst_hbm.at[pl.ds(e0 + i * CHUNK + h * HALF, HALF)],
            dstb.at[j, h], dsem[j]).wait()

    def issue_gather(i, b):
      pltpu.async_copy(
          xbf_hbm.at[src_all.at[pl.ds(i * CHUNK, CHUNK)]], rows_bf.at[b],
          gsem[b])

    def wait_gather(i, b):
      pltpu.make_async_copy(
          xbf_hbm.at[src_all.at[pl.ds(i * CHUNK, CHUNK)]], rows_bf.at[b],
          gsem[b]).wait()

    def convert(b, h):
      def conv_row(r, _):
        for g in range(D // 32):
          w = rows_bf[b, h * HALF + r, pl.ds(16 * g, 16)]
          fb[h, r, pl.ds(32 * g, 16)] = plsc.bitcast(w << 16, jnp.float32)
          fb[h, r, pl.ds(32 * g + 16, 16)] = plsc.bitcast(
              w & (-65536), jnp.float32)
        return 0
      lax.fori_loop(0, HALF, conv_row, 0)

    def issue_scatter(j, h):
      pltpu.async_copy(fb.at[h], acc_s.at[dstb.at[j, h]], fsem[h], add=True)
      if with_cnt:
        pltpu.async_copy(ones_v, cnt_s.at[dstb.at[j, h]], csem[h], add=True)

    def wait_scatter(j, h):
      pltpu.make_async_copy(fb.at[h], acc_s.at[dstb.at[j, h]], fsem[h]).wait()
      if with_cnt:
        pltpu.make_async_copy(ones_v, cnt_s.at[dstb.at[j, h]], csem[h]).wait()

    bulk = pltpu.async_copy(
        src_hbm.at[pl.ds(e0, NCHUNK * CHUNK)], src_all, isem)
    pf_dst(0, 0)
    pf_dst(1, 1)

    def zrow(i, _):
      for j in range(D // 16):
        zbuf_v[i, pl.ds(j * 16, 16)] = jnp.zeros((16,), jnp.float32)
      return 0
    lax.fori_loop(0, ZROWS, zrow, 0)

    def zacc(k, _):
      pltpu.sync_copy(zbuf_v, acc_s.at[pl.ds(sid * ROWS_PT + k * ZROWS, ZROWS)])
      return 0
    lax.fori_loop(0, ROWS_PT // ZROWS, zacc, 0)

    if with_cnt:
      def zone(i, _):
        ones_v[pl.ds(i * 16, 16)] = jnp.ones((16,), jnp.float32)
        return 0
      lax.fori_loop(0, HALF // 16, zone, 0)
      onest[pl.ds(0, 16)] = jnp.ones((16,), jnp.float32)

      def zcnt(i, _):
        zcnt_v[pl.ds(i * 16, 16)] = jnp.zeros((16,), jnp.float32)
        return 0
      lax.fori_loop(0, CNT_PT // 16, zcnt, 0)
      pltpu.sync_copy(zcnt_v, cnt_s.at[pl.ds(sid * CNT_PT, CNT_PT)])

    bulk.wait()
    plsc.subcore_barrier()

    issue_gather(0, 0)

    def sub_iter(i, b, j, first, last, prefetch):
      if not last:
        issue_gather(i + 1, 1 - b)
      wait_gather(i, b)
      wait_dst(i, j)
      for h in (0, 1):
        if not first:
          wait_scatter((j - 1) % 3, h)
        convert(b, h)
        issue_scatter(j, h)
      if prefetch:
        pf_dst(i + 2, (j + 2) % 3)

    sub_iter(0, 0, 0, True, False, True)
    sub_iter(1, 1, 1, False, False, True)

    def steady(g, _):
      for u in range(6):
        i = 2 + 6 * g + u
        sub_iter(i, u % 2, (2 + u) % 3, False, False, True)
      return 0
    lax.fori_loop(0, 12, steady, 0)

    for i in range(74, NCHUNK):
      sub_iter(i, i % 2, i % 3, False, i == NCHUNK - 1, i + 2 < NCHUNK)

    jl = (NCHUNK - 1) % 3
    pltpu.sync_copy(src_hbm.at[pl.ds(e0 + NCHUNK * CHUNK, TAIL)], srct)
    pltpu.sync_copy(dst_hbm.at[pl.ds(e0 + NCHUNK * CHUNK, TAIL)], dstt)
    pltpu.async_copy(xbf_hbm.at[srct], tailbf, gsem[0]).wait()

    def conv_tail(r, _):
      for g in range(D // 32):
        w = tailbf[r, pl.ds(16 * g, 16)]
        rowst[r, pl.ds(32 * g, 16)] = plsc.bitcast(w << 16, jnp.float32)
        rowst[r, pl.ds(32 * g + 16, 16)] = plsc.bitcast(
            w & (-65536), jnp.float32)
      return 0
    lax.fori_loop(0, TAIL, conv_tail, 0)
    for h in (0, 1):
      wait_scatter(jl, h)
    pltpu.sync_copy(rowst, acc_s.at[dstt], add=True)
    if with_cnt:
      pltpu.sync_copy(onest, cnt_s.at[dstt], add=True)

    plsc.subcore_barrier()

    def out_step(k, _):
      r0 = sid * ROWS_PT + k * ZROWS
      pltpu.sync_copy(acc_s.at[pl.ds(r0, ZROWS)], zbuf_v)
      pltpu.sync_copy(zbuf_v, sum_hbm.at[cid, pl.ds(r0, ZROWS)])
      return 0
    lax.fori_loop(0, ROWS_PT // ZROWS, out_step, 0)

    if with_cnt:
      pltpu.sync_copy(cnt_s.at[pl.ds(sid * CNT_PT, CNT_PT)], zcnt_v)
      pltpu.sync_copy(
          zcnt_v, cnt_hbm.at[pl.ds(cid * CNT_PAD + sid * CNT_PT, CNT_PT)])

  return pl.kernel(
      body, out_type=out_type, mesh=mesh, scratch_types=scratch,
      compiler_params=pltpu.CompilerParams(
          use_tc_tiling_on_sc=False, needs_layout_passes=False))


_sc_agg_cnt = _make_sc_agg(True)
_sc_agg = _make_sc_agg(False)

BT = 1000


def _make_tc_dense(relu: bool):
  def body(sum_ref, cnt_ref, x_ref, wl_ref, bl_ref, wr_ref, o_ref):
    s = sum_ref[0] + sum_ref[1]
    c = cnt_ref[0] + cnt_ref[1]
    agg = s / jnp.maximum(c, 1.0)
    out = jnp.dot(agg, wl_ref[...], preferred_element_type=jnp.float32)
    out += bl_ref[...]
    out += jnp.dot(x_ref[...], wr_ref[...], preferred_element_type=jnp.float32)
    nrm = jnp.sqrt(jnp.sum(out * out, axis=1, keepdims=True))
    out = out / jnp.maximum(nrm, 1e-12)
    if relu:
      out = jnp.maximum(out, 0.0)
    o_ref[...] = out

  grid = N_NODES // BT
  return pl.pallas_call(
      body,
      grid=(grid,),
      in_specs=[
          pl.BlockSpec((NC, BT, D), lambda i: (0, i, 0)),
          pl.BlockSpec((NC, BT, 1), lambda i: (0, i, 0)),
          pl.BlockSpec((BT, D), lambda i: (i, 0)),
          pl.BlockSpec((D, D), lambda i: (0, 0)),
          pl.BlockSpec((1, D), lambda i: (0, 0)),
          pl.BlockSpec((D, D), lambda i: (0, 0)),
      ],
      out_specs=pl.BlockSpec((BT, D), lambda i: (i, 0)),
      out_shape=jax.ShapeDtypeStruct((N_NODES, D), jnp.float32),
  )


_tc_dense_relu = _make_tc_dense(True)
_tc_dense = _make_tc_dense(False)


def kernel(x, edge_index, edge_attr, W1l, b1, W1r, W2l, b2, W2r):
  del edge_attr
  src = edge_index[0].astype(jnp.int32)
  dst = edge_index[1].astype(jnp.int32)

  psum1, pcnt = _sc_agg_cnt(_shuffled_bf16(x), src, dst)
  cnt = pcnt.reshape(NC, CNT_PAD, 1)
  h1 = _tc_dense_relu(psum1, cnt, x, W1l, b1.reshape(1, D), W1r)
  (psum2,) = _sc_agg(_shuffled_bf16(h1), src, dst)
  h2 = _tc_dense(psum2, cnt, h1, W2l, b2.reshape(1, D), W2r)
  return h2


# speedup vs baseline: 1.8054x; 1.8054x over previous
import jax
import jax.numpy as jnp
from jax import lax
from jax.experimental import pallas as pl
from jax.experimental.pallas import tpu as pltpu
from jax.experimental.pallas import tpu_sc as plsc

N_NODES = 10000
N_EDGES = 320000
D = 128

NC = 2
NS = 16
NW = NC * NS
EPW = N_EDGES // NW
CHUNK = 128
NCHUNK = EPW // CHUNK
TAIL = EPW - NCHUNK * CHUNK
N_PAD = 10240
ROWS_PT = N_PAD // NS
ZROWS = CHUNK
CNT_PAD = N_PAD
CNT_PT = CNT_PAD // NS


def _make_sc_agg(with_cnt: bool):
  mesh = plsc.VectorSubcoreMesh(
      core_axis_name="c", subcore_axis_name="s", num_cores=NC, num_subcores=NS)
  out_type = [jax.ShapeDtypeStruct((NC, N_PAD, D), jnp.float32)]
  scratch = [
      pltpu.VMEM((NCHUNK * CHUNK,), jnp.int32),
      pltpu.VMEM((3, CHUNK), jnp.int32),
      pltpu.VMEM((2, CHUNK, D), jnp.float32),
      pltpu.VMEM((TAIL,), jnp.int32),
      pltpu.VMEM((TAIL,), jnp.int32),
      pltpu.VMEM((TAIL, D), jnp.float32),
      pltpu.VMEM_SHARED((N_PAD, D), jnp.float32),
      pltpu.SemaphoreType.DMA,
      pltpu.SemaphoreType.DMA,
      pltpu.SemaphoreType.DMA,
      pltpu.SemaphoreType.DMA,
      pltpu.SemaphoreType.DMA,
      pltpu.SemaphoreType.DMA,
      pltpu.SemaphoreType.DMA,
      pltpu.SemaphoreType.DMA,
  ]
  if with_cnt:
    out_type.append(jax.ShapeDtypeStruct((NC * CNT_PAD,), jnp.float32))
    scratch += [
        pltpu.VMEM((CHUNK,), jnp.float32),
        pltpu.VMEM((TAIL,), jnp.float32),
        pltpu.VMEM((CNT_PT,), jnp.float32),
        pltpu.VMEM_SHARED((CNT_PAD,), jnp.float32),
        pltpu.SemaphoreType.DMA,
        pltpu.SemaphoreType.DMA,
    ]

  def body(x_hbm, src_hbm, dst_hbm, sum_hbm, *rest):
    if with_cnt:
      (cnt_hbm, src_all, dstb, rows_v, srct, dstt, rowst, acc_s,
       isem, gsem0, gsem1, ssem0, ssem1, dsem0, dsem1, dsem2,
       ones_v, onest, zcnt_v, cnt_s, csem0, csem1) = rest
    else:
      (src_all, dstb, rows_v, srct, dstt, rowst, acc_s,
       isem, gsem0, gsem1, ssem0, ssem1, dsem0, dsem1, dsem2) = rest
    gsem = (gsem0, gsem1)
    ssem = (ssem0, ssem1)
    dsem = (dsem0, dsem1, dsem2)
    if with_cnt:
      csem = (csem0, csem1)
    zbuf_v = rows_v.at[0]
    cid = lax.axis_index("c")
    sid = lax.axis_index("s")
    wid = sid * NC + cid
    e0 = wid * EPW

    def pf_dst(i, j):
      pltpu.async_copy(dst_hbm.at[pl.ds(e0 + i * CHUNK, CHUNK)],
                       dstb.at[j], dsem[j])

    def wait_dst(i, j):
      pltpu.make_async_copy(dst_hbm.at[pl.ds(e0 + i * CHUNK, CHUNK)],
                            dstb.at[j], dsem[j]).wait()

    def issue_gather(i, b):
      pltpu.async_copy(
          x_hbm.at[src_all.at[pl.ds(i * CHUNK, CHUNK)]], rows_v.at[b], gsem[b])

    def wait_gather(i, b):
      pltpu.make_async_copy(
          x_hbm.at[src_all.at[pl.ds(i * CHUNK, CHUNK)]], rows_v.at[b],
          gsem[b]).wait()

    def issue_scatter(b, j):
      pltpu.async_copy(rows_v.at[b], acc_s.at[dstb.at[j]], ssem[b], add=True)
      if with_cnt:
        pltpu.async_copy(ones_v, cnt_s.at[dstb.at[j]], csem[b], add=True)

    def wait_scatter(b, j):
      pltpu.make_async_copy(rows_v.at[b], acc_s.at[dstb.at[j]], ssem[b]).wait()
      if with_cnt:
        pltpu.make_async_copy(ones_v, cnt_s.at[dstb.at[j]], csem[b]).wait()

    bulk = pltpu.async_copy(
        src_hbm.at[pl.ds(e0, NCHUNK * CHUNK)], src_all, isem)
    pf_dst(0, 0)
    pf_dst(1, 1)

    def zrow(i, _):
      for j in range(D // 16):
        zbuf_v[i, pl.ds(j * 16, 16)] = jnp.zeros((16,), jnp.float32)
      return 0
    lax.fori_loop(0, ZROWS, zrow, 0)

    def zacc(k, _):
      pltpu.sync_copy(zbuf_v, acc_s.at[pl.ds(sid * ROWS_PT + k * ZROWS, ZROWS)])
      return 0
    lax.fori_loop(0, ROWS_PT // ZROWS, zacc, 0)

    if with_cnt:
      def zone(i, _):
        ones_v[pl.ds(i * 16, 16)] = jnp.ones((16,), jnp.float32)
        return 0
      lax.fori_loop(0, CHUNK // 16, zone, 0)
      onest[pl.ds(0, 16)] = jnp.ones((16,), jnp.float32)

      def zcnt(i, _):
        zcnt_v[pl.ds(i * 16, 16)] = jnp.zeros((16,), jnp.float32)
        return 0
      lax.fori_loop(0, CNT_PT // 16, zcnt, 0)
      pltpu.sync_copy(zcnt_v, cnt_s.at[pl.ds(sid * CNT_PT, CNT_PT)])

    bulk.wait()
    plsc.subcore_barrier()

    issue_gather(0, 0)
    pf_dst(2, 2)
    issue_gather(1, 1)
    wait_gather(0, 0)
    wait_dst(0, 0)
    issue_scatter(0, 0)

    def sub_iter(i, b, j, jp, jm, prefetch):
      wait_scatter(b, jp)
      issue_gather(i, b)
      if prefetch:
        pf_dst(i + 1, jp)
      wait_gather(i - 1, 1 - b)
      wait_dst(i - 1, jm)
      issue_scatter(1 - b, jm)

    def steady(g, _):
      for u in range(6):
        i = 2 + 6 * g + u
        b = u % 2
        j = (2 + u) % 3
        sub_iter(i, b, j, (j + 1) % 3, (j + 2) % 3, True)
      return 0
    lax.fori_loop(0, 12, steady, 0)

    for i in range(74, NCHUNK):
      b = i % 2
      j = i % 3
      sub_iter(i, b, j, (j + 1) % 3, (j + 2) % 3, i + 1 < NCHUNK)

    i = NCHUNK
    b, jm = i % 2, (i - 1) % 3
    wait_scatter(b, (i % 3 + 1) % 3)
    wait_gather(i - 1, 1 - b)
    wait_dst(i - 1, jm)
    issue_scatter(1 - b, jm)

    pltpu.sync_copy(src_hbm.at[pl.ds(e0 + NCHUNK * CHUNK, TAIL)], srct)
    pltpu.sync_copy(dst_hbm.at[pl.ds(e0 + NCHUNK * CHUNK, TAIL)], dstt)
    pltpu.async_copy(x_hbm.at[srct], rowst, gsem[b]).wait()
    pltpu.sync_copy(rowst, acc_s.at[dstt], add=True)
    if with_cnt:
      pltpu.sync_copy(onest, cnt_s.at[dstt], add=True)
    wait_scatter(1 - b, jm)

    plsc.subcore_barrier()

    def out_step(k, _):
      r0 = sid * ROWS_PT + k * ZROWS
      pltpu.sync_copy(acc_s.at[pl.ds(r0, ZROWS)], zbuf_v)
      pltpu.sync_copy(zbuf_v, sum_hbm.at[cid, pl.ds(r0, ZROWS)])
      return 0
    lax.fori_loop(0, ROWS_PT // ZROWS, out_step, 0)

    if with_cnt:
      pltpu.sync_copy(cnt_s.at[pl.ds(sid * CNT_PT, CNT_PT)], zcnt_v)
      pltpu.sync_copy(
          zcnt_v, cnt_hbm.at[pl.ds(cid * CNT_PAD + sid * CNT_PT, CNT_PT)])

  return pl.kernel(body, out_type=out_type, mesh=mesh, scratch_types=scratch)


_sc_agg_cnt = _make_sc_agg(True)
_sc_agg = _make_sc_agg(False)

BT = 1000


def _make_tc_dense(relu: bool):
  def body(sum_ref, cnt_ref, x_ref, wl_ref, bl_ref, wr_ref, o_ref):
    s = sum_ref[0] + sum_ref[1]
    c = cnt_ref[0] + cnt_ref[1]
    agg = s / jnp.maximum(c, 1.0)
    out = jnp.dot(agg, wl_ref[...], preferred_element_type=jnp.float32)
    out += bl_ref[...]
    out += jnp.dot(x_ref[...], wr_ref[...], preferred_element_type=jnp.float32)
    nrm = jnp.sqrt(jnp.sum(out * out, axis=1, keepdims=True))
    out = out / jnp.maximum(nrm, 1e-12)
    if relu:
      out = jnp.maximum(out, 0.0)
    o_ref[...] = out

  grid = N_NODES // BT
  return pl.pallas_call(
      body,
      grid=(grid,),
      in_specs=[
          pl.BlockSpec((NC, BT, D), lambda i: (0, i, 0)),
          pl.BlockSpec((NC, BT, 1), lambda i: (0, i, 0)),
          pl.BlockSpec((BT, D), lambda i: (i, 0)),
          pl.BlockSpec((D, D), lambda i: (0, 0)),
          pl.BlockSpec((1, D), lambda i: (0, 0)),
          pl.BlockSpec((D, D), lambda i: (0, 0)),
      ],
      out_specs=pl.BlockSpec((BT, D), lambda i: (i, 0)),
      out_shape=jax.ShapeDtypeStruct((N_NODES, D), jnp.float32),
  )


_tc_dense_relu = _make_tc_dense(True)
_tc_dense = _make_tc_dense(False)


def kernel(x, edge_index, edge_attr, W1l, b1, W1r, W2l, b2, W2r):
  del edge_attr
  src = edge_index[0].astype(jnp.int32)
  dst = edge_index[1].astype(jnp.int32)

  psum1, pcnt = _sc_agg_cnt(x, src, dst)
  cnt = pcnt.reshape(NC, CNT_PAD, 1)
  h1 = _tc_dense_relu(psum1, cnt, x, W1l, b1.reshape(1, D), W1r)
  (psum2,) = _sc_agg(h1, src, dst)
  h2 = _tc_dense(psum2, cnt, h1, W2l, b2.reshape(1, D), W2r)
  return h2


# speedup vs baseline: 1.8651x; 1.0330x over previous
import jax
import jax.numpy as jnp
from jax import lax
from jax.experimental import pallas as pl
from jax.experimental.pallas import tpu as pltpu
from jax.experimental.pallas import tpu_sc as plsc

N_NODES = 10000
N_EDGES = 320000
D = 128

NC = 2
NS = 16
NW = NC * NS
EPW = N_EDGES // NW
CHUNK = 128
NCHUNK = EPW // CHUNK
TAIL = EPW - NCHUNK * CHUNK
N_PAD = 10240
ROWS_PT = N_PAD // NS
ZROWS = CHUNK
CNT_PAD = N_PAD
CNT_PT = CNT_PAD // NS


def _make_sc_agg(with_cnt: bool):
  mesh = plsc.VectorSubcoreMesh(
      core_axis_name="c", subcore_axis_name="s", num_cores=NC, num_subcores=NS)
  out_type = [jax.ShapeDtypeStruct((NC, N_PAD, D), jnp.float32)]
  scratch = [
      pltpu.VMEM((NCHUNK * CHUNK,), jnp.int32),
      pltpu.VMEM((3, CHUNK), jnp.int32),
      pltpu.VMEM((2, CHUNK, D), jnp.float32),
      pltpu.VMEM((TAIL,), jnp.int32),
      pltpu.VMEM((TAIL,), jnp.int32),
      pltpu.VMEM((TAIL, D), jnp.float32),
      pltpu.VMEM_SHARED((N_PAD, D), jnp.float32),
      pltpu.SemaphoreType.DMA,
      pltpu.SemaphoreType.DMA,
      pltpu.SemaphoreType.DMA,
      pltpu.SemaphoreType.DMA,
      pltpu.SemaphoreType.DMA,
      pltpu.SemaphoreType.DMA,
      pltpu.SemaphoreType.DMA,
      pltpu.SemaphoreType.DMA,
  ]
  if with_cnt:
    out_type.append(jax.ShapeDtypeStruct((NC * CNT_PAD,), jnp.float32))
    scratch += [
        pltpu.VMEM((CHUNK,), jnp.float32),
        pltpu.VMEM((TAIL,), jnp.float32),
        pltpu.VMEM((CNT_PT,), jnp.float32),
        pltpu.VMEM_SHARED((CNT_PAD,), jnp.float32),
        pltpu.SemaphoreType.DMA,
        pltpu.SemaphoreType.DMA,
    ]

  def body(x_hbm, ei_hbm, sum_hbm, *rest):
    src_hbm = ei_hbm.at[0, 0]
    dst_hbm = ei_hbm.at[1, 0]
    if with_cnt:
      (cnt_hbm, src_all, dstb, rows_v, srct, dstt, rowst, acc_s,
       isem, gsem0, gsem1, ssem0, ssem1, dsem0, dsem1, dsem2,
       ones_v, onest, zcnt_v, cnt_s, csem0, csem1) = rest
    else:
      (src_all, dstb, rows_v, srct, dstt, rowst, acc_s,
       isem, gsem0, gsem1, ssem0, ssem1, dsem0, dsem1, dsem2) = rest
    gsem = (gsem0, gsem1)
    ssem = (ssem0, ssem1)
    dsem = (dsem0, dsem1, dsem2)
    if with_cnt:
      csem = (csem0, csem1)
    zbuf_v = rows_v.at[0]
    cid = lax.axis_index("c")
    sid = lax.axis_index("s")
    wid = sid * NC + cid
    e0 = wid * EPW

    def pf_dst(i, j):
      pltpu.async_copy(dst_hbm.at[pl.ds(e0 + i * CHUNK, CHUNK)],
                       dstb.at[j], dsem[j])

    def wait_dst(i, j):
      pltpu.make_async_copy(dst_hbm.at[pl.ds(e0 + i * CHUNK, CHUNK)],
                            dstb.at[j], dsem[j]).wait()

    def issue_gather(i, b):
      pltpu.async_copy(
          x_hbm.at[src_all.at[pl.ds(i * CHUNK, CHUNK)]], rows_v.at[b], gsem[b])

    def wait_gather(i, b):
      pltpu.make_async_copy(
          x_hbm.at[src_all.at[pl.ds(i * CHUNK, CHUNK)]], rows_v.at[b],
          gsem[b]).wait()

    def issue_scatter(b, j):
      pltpu.async_copy(rows_v.at[b], acc_s.at[dstb.at[j]], ssem[b], add=True)
      if with_cnt:
        pltpu.async_copy(ones_v, cnt_s.at[dstb.at[j]], csem[b], add=True)

    def wait_scatter(b, j):
      pltpu.make_async_copy(rows_v.at[b], acc_s.at[dstb.at[j]], ssem[b]).wait()
      if with_cnt:
        pltpu.make_async_copy(ones_v, cnt_s.at[dstb.at[j]], csem[b]).wait()

    bulk = pltpu.async_copy(
        src_hbm.at[pl.ds(e0, NCHUNK * CHUNK)], src_all, isem)
    pf_dst(0, 0)
    pf_dst(1, 1)

    def zrow(i, _):
      for j in range(D // 16):
        zbuf_v[i, pl.ds(j * 16, 16)] = jnp.zeros((16,), jnp.float32)
      return 0
    lax.fori_loop(0, ZROWS, zrow, 0)

    def zacc(k, _):
      pltpu.sync_copy(zbuf_v, acc_s.at[pl.ds(sid * ROWS_PT + k * ZROWS, ZROWS)])
      return 0
    lax.fori_loop(0, ROWS_PT // ZROWS, zacc, 0)

    if with_cnt:
      def zone(i, _):
        ones_v[pl.ds(i * 16, 16)] = jnp.ones((16,), jnp.float32)
        return 0
      lax.fori_loop(0, CHUNK // 16, zone, 0)
      onest[pl.ds(0, 16)] = jnp.ones((16,), jnp.float32)

      def zcnt(i, _):
        zcnt_v[pl.ds(i * 16, 16)] = jnp.zeros((16,), jnp.float32)
        return 0
      lax.fori_loop(0, CNT_PT // 16, zcnt, 0)
      pltpu.sync_copy(zcnt_v, cnt_s.at[pl.ds(sid * CNT_PT, CNT_PT)])

    bulk.wait()
    plsc.subcore_barrier()

    issue_gather(0, 0)
    pf_dst(2, 2)
    issue_gather(1, 1)
    wait_gather(0, 0)
    wait_dst(0, 0)
    issue_scatter(0, 0)

    def sub_iter(i, b, j, jp, jm, prefetch):
      wait_scatter(b, jp)
      issue_gather(i, b)
      if prefetch:
        pf_dst(i + 1, jp)
      wait_gather(i - 1, 1 - b)
      wait_dst(i - 1, jm)
      issue_scatter(1 - b, jm)

    def steady(g, _):
      for u in range(6):
        i = 2 + 6 * g + u
        b = u % 2
        j = (2 + u) % 3
        sub_iter(i, b, j, (j + 1) % 3, (j + 2) % 3, True)
      return 0
    lax.fori_loop(0, 12, steady, 0)

    for i in range(74, NCHUNK):
      b = i % 2
      j = i % 3
      sub_iter(i, b, j, (j + 1) % 3, (j + 2) % 3, i + 1 < NCHUNK)

    i = NCHUNK
    b, jm = i % 2, (i - 1) % 3
    wait_scatter(b, (i % 3 + 1) % 3)
    wait_gather(i - 1, 1 - b)
    wait_dst(i - 1, jm)
    issue_scatter(1 - b, jm)

    pltpu.sync_copy(src_hbm.at[pl.ds(e0 + NCHUNK * CHUNK, TAIL)], srct)
    pltpu.sync_copy(dst_hbm.at[pl.ds(e0 + NCHUNK * CHUNK, TAIL)], dstt)
    pltpu.async_copy(x_hbm.at[srct], rowst, gsem[b]).wait()
    pltpu.sync_copy(rowst, acc_s.at[dstt], add=True)
    if with_cnt:
      pltpu.sync_copy(onest, cnt_s.at[dstt], add=True)
    wait_scatter(1 - b, jm)

    plsc.subcore_barrier()

    r0 = sid * ROWS_PT
    pltpu.sync_copy(acc_s.at[pl.ds(r0, ROWS_PT)],
                    sum_hbm.at[cid, pl.ds(r0, ROWS_PT)])
    if with_cnt:
      pltpu.sync_copy(
          cnt_s.at[pl.ds(sid * CNT_PT, CNT_PT)],
          cnt_hbm.at[pl.ds(cid * CNT_PAD + sid * CNT_PT, CNT_PT)])

  return pl.kernel(body, out_type=out_type, mesh=mesh, scratch_types=scratch)


_sc_agg_cnt = _make_sc_agg(True)
_sc_agg = _make_sc_agg(False)

BT = 1000


def _make_tc_dense(relu: bool):
  def body(sum_ref, cnt_ref, x_ref, wl_ref, bl_ref, wr_ref, o_ref):
    s = sum_ref[0] + sum_ref[1]
    c = cnt_ref[0] + cnt_ref[1]
    agg = s / jnp.maximum(c, 1.0)
    out = jnp.dot(agg, wl_ref[...], preferred_element_type=jnp.float32)
    out += bl_ref[...]
    out += jnp.dot(x_ref[...], wr_ref[...], preferred_element_type=jnp.float32)
    nrm = jnp.sqrt(jnp.sum(out * out, axis=1, keepdims=True))
    out = out / jnp.maximum(nrm, 1e-12)
    if relu:
      out = jnp.maximum(out, 0.0)
    o_ref[...] = out

  grid = N_NODES // BT
  return pl.pallas_call(
      body,
      grid=(grid,),
      in_specs=[
          pl.BlockSpec((NC, BT, D), lambda i: (0, i, 0)),
          pl.BlockSpec((NC, BT, 1), lambda i: (0, i, 0)),
          pl.BlockSpec((BT, D), lambda i: (i, 0)),
          pl.BlockSpec((D, D), lambda i: (0, 0)),
          pl.BlockSpec((1, D), lambda i: (0, 0)),
          pl.BlockSpec((D, D), lambda i: (0, 0)),
      ],
      out_specs=pl.BlockSpec((BT, D), lambda i: (i, 0)),
      out_shape=jax.ShapeDtypeStruct((N_NODES, D), jnp.float32),
  )


_tc_dense_relu = _make_tc_dense(True)
_tc_dense = _make_tc_dense(False)


def kernel(x, edge_index, edge_attr, W1l, b1, W1r, W2l, b2, W2r):
  del edge_attr
  ei = edge_index.astype(jnp.int32).reshape(2, 1, N_EDGES)

  psum1, pcnt = _sc_agg_cnt(x, ei)
  cnt = pcnt.reshape(NC, CNT_PAD, 1)
  h1 = _tc_dense_relu(psum1, cnt, x, W1l, b1.reshape(1, D), W1r)
  (psum2,) = _sc_agg(h1, ei)
  h2 = _tc_dense(psum2, cnt, h1, W2l, b2.reshape(1, D), W2r)
  return h2
